# baseline (speedup 1.0000x reference)
import functools

import jax
import jax.numpy as jnp
from jax import lax
from jax.experimental import pallas as pl
from jax.experimental.pallas import tpu as pltpu
from jax.experimental.pallas import tpu_sc as plsc

N = 10000
E = 320000
D_EDGE = 16
H = 128
C = 2
G = 64
BN_EPS = 1e-5

NC = 2
NS = 16
NW = NC * NS
EDGES_PER_TILE = E // NW
CH = 40
NCHUNK = EDGES_PER_TILE // CH
NSUPER = NCHUNK // 2
NP = 10240
ROWS_PER_TILE = NP // NS
ZR = 128
HV = H // 16
EW = H // 2


def _perm_matrix():
    col = lax.broadcasted_iota(jnp.int32, (H, H), 1)
    row = lax.broadcasted_iota(jnp.int32, (H, H), 0)
    src = (col // 32) * 32 + (col % 2) * 16 + (col % 32) // 2
    return (row == src).astype(jnp.float32)


def _edge_feats_body(attr_ref, Wee_ref, bee_ref, Wle_ref, ble_ref, e_ref):
    attr = attr_ref[...]
    P = _perm_matrix()
    Wc = jnp.dot(jnp.dot(Wee_ref[...], Wle_ref[...],
                         preferred_element_type=jnp.float32), P,
                 preferred_element_type=jnp.float32)
    bc = jnp.dot(jnp.dot(bee_ref[...], Wle_ref[...],
                         preferred_element_type=jnp.float32) + ble_ref[...], P,
                 preferred_element_type=jnp.float32)
    e_ref[...] = (jnp.dot(attr, Wc, preferred_element_type=jnp.float32)
                  + bc).astype(jnp.bfloat16)


def _edge_feats(edge_attr, W_ee, b_ee, Wle, ble):
    BE = 4000
    grid = E // BE
    full = lambda shape: pl.BlockSpec(shape, lambda i: (0, 0))
    return pl.pallas_call(
        _edge_feats_body,
        grid=(grid,),
        in_specs=[
            pl.BlockSpec((BE, D_EDGE), lambda i: (i, 0)),
            full((D_EDGE, H)), full((1, H)),
            full((H, H)), full((1, H)),
        ],
        out_specs=pl.BlockSpec((BE, H), lambda i: (i, 0)),
        out_shape=jax.ShapeDtypeStruct((E, H), jnp.bfloat16),
    )(edge_attr, W_ee, b_ee.reshape(1, H), Wle, ble.reshape(1, H))


def _sc_agg_body(x_hbm, src_hbm, dst_hbm, e_hbm, out_hbm,
                 src_v, dst_v, rows_v, e_v, m_v, z_v, acc,
                 sem_src, sem_dst, sem_g, sem_e):
    c = lax.axis_index("c")
    s = lax.axis_index("s")
    wid = c * NS + s

    zeros16 = jnp.zeros((16,), jnp.float32)

    def zfill(i, _):
        r = i // HV
        col = (i % HV) * 16
        z_v[r, pl.ds(col, 16)] = zeros16
        return 0

    lax.fori_loop(0, ZR * HV, zfill, 0)

    def zcopy(j, _):
        pltpu.sync_copy(z_v, acc.at[pl.ds(s * ROWS_PER_TILE + j * ZR, ZR)])
        return 0

    lax.fori_loop(0, ROWS_PER_TILE // ZR, zcopy, 0)
    plsc.subcore_barrier()

    base0 = wid * EDGES_PER_TILE

    def issue_src(ci, b):
        pltpu.async_copy(src_hbm.at[pl.ds(base0 + ci * CH, CH)], src_v[b],
                         sem_src[b])

    def issue_chunk(ci, b):
        pltpu.async_copy(x_hbm.at[src_v[b]], rows_v[b], sem_g[b])
        pltpu.async_copy(e_hbm.at[pl.ds((base0 + ci * CH) * EW, CH * EW)],
                         e_v[b], sem_e[b])
        pltpu.async_copy(dst_hbm.at[pl.ds(base0 + ci * CH, CH)], dst_v[b],
                         sem_dst[b])

    def wait_idx(sem, ref):
        pltpu.make_async_copy(src_hbm.at[pl.ds(0, CH)], ref, sem).wait()

    def wait_row(sem, ref):
        pltpu.make_async_copy(e_hbm.at[pl.ds(0, CH * EW)], ref, sem).wait()

    def stage(k, cur, ci, has_next, has_next2):
        nxt = 1 - cur

        @pl.when(has_next)
        def _():
            wait_idx(sem_src[nxt], src_v[nxt])
            issue_chunk(ci + 1, nxt)

        pltpu.make_async_copy(x_hbm.at[src_v[cur]], rows_v[cur],
                              sem_g[cur]).wait()
        wait_row(sem_e[cur], e_v[cur])

        @pl.when(has_next2)
        def _():
            issue_src(ci + 2, cur)

        @plsc.parallel_loop(0, CH, 1, unroll=2)
        def _compute(i):
            for g in range(4):
                w = e_v[cur][pl.ds(i * EW + g * 16, 16)]
                a = lax.bitcast_convert_type(w << 16, jnp.float32)
                b = lax.bitcast_convert_type(w & jnp.int32(-65536), jnp.float32)
                x0 = rows_v[cur][i, pl.ds(g * 32, 16)]
                x1 = rows_v[cur][i, pl.ds(g * 32 + 16, 16)]
                m_v[cur][i, pl.ds(g * 32, 16)] = jnp.maximum(x0 + a, 0.0)
                m_v[cur][i, pl.ds(g * 32 + 16, 16)] = jnp.maximum(x1 + b, 0.0)

        wait_idx(sem_dst[cur], dst_v[cur])
        pltpu.sync_copy(m_v[cur], acc.at[dst_v[cur]], add=True)

    issue_src(0, 0)
    wait_idx(sem_src[0], src_v[0])
    issue_chunk(0, 0)
    issue_src(1, 1)

    def super_step(k, _):
        t = jnp.bool_(True)
        stage(k, 0, 2 * k, t, k < NSUPER - 1)
        stage(k, 1, 2 * k + 1, k < NSUPER - 1, k < NSUPER - 1)
        return 0

    lax.fori_loop(0, NSUPER, super_step, 0)
    plsc.subcore_barrier()
    pltpu.sync_copy(acc.at[pl.ds(s * ROWS_PER_TILE, ROWS_PER_TILE)],
                    out_hbm.at[c, pl.ds(s * ROWS_PER_TILE, ROWS_PER_TILE)])


def _sc_agg(x, src, dst, e):
    mesh = plsc.VectorSubcoreMesh(core_axis_name="c", subcore_axis_name="s",
                                  num_cores=NC, num_subcores=NS)
    idx2 = [pltpu.VMEM((CH,), jnp.int32)] * 2
    buf2 = [pltpu.VMEM((CH, H), jnp.float32)] * 2
    ebuf2 = [pltpu.VMEM((CH * EW,), jnp.int32)] * 2
    sem2 = [pltpu.SemaphoreType.DMA] * 2
    k = functools.partial(
        pl.kernel,
        out_type=jax.ShapeDtypeStruct((NC, NP, H), jnp.float32),
        mesh=mesh,
        scratch_types=[
            idx2, idx2, buf2, ebuf2, buf2,
            pltpu.VMEM((ZR, H), jnp.float32),
            pltpu.VMEM_SHARED((NP, H), jnp.float32),
            sem2, sem2, sem2, sem2,
        ],
    )(_sc_agg_body)
    e_i32 = jax.lax.bitcast_convert_type(e.reshape(E, EW, 2), jnp.int32)
    return k(x, src, dst, e_i32.reshape(E * EW))


def _mlp_body(x_ref, agg_ref, W1_ref, b1_ref, W2_ref, b2_ref, sc_ref, be_ref,
              out_ref):
    h = x_ref[...] + agg_ref[0] + agg_ref[1]
    h1 = jnp.maximum(jnp.dot(h, W1_ref[...], preferred_element_type=jnp.float32)
                     + b1_ref[...], 0.0)
    h2 = jnp.dot(h1, W2_ref[...], preferred_element_type=jnp.float32) + b2_ref[...]
    out_ref[...] = jnp.maximum(h2 * sc_ref[...] + be_ref[...], 0.0)


def _mlp(x, agg, W1, b1, W2, b2, g, be):
    BN = 1000
    grid = N // BN
    scale = (g / jnp.sqrt(1.0 + BN_EPS)).reshape(1, H)
    full = lambda shape: pl.BlockSpec(shape, lambda i: (0, 0))
    return pl.pallas_call(
        _mlp_body,
        grid=(grid,),
        in_specs=[
            pl.BlockSpec((BN, H), lambda i: (i, 0)),
            pl.BlockSpec((NC, BN, H), lambda i: (0, i, 0)),
            full((H, H)), full((1, H)),
            full((H, H)), full((1, H)),
            full((1, H)), full((1, H)),
        ],
        out_specs=pl.BlockSpec((BN, H), lambda i: (i, 0)),
        out_shape=jax.ShapeDtypeStruct((N, H), jnp.float32),
    )(x, agg, W1, b1.reshape(1, H), W2, b2.reshape(1, H), scale,
      be.reshape(1, H))


def _pool_body(h_ref, batch_ref, Wc_ref, bc_ref, out_ref, acc_ref):
    i = pl.program_id(0)

    @pl.when(i == 0)
    def _():
        acc_ref[...] = jnp.zeros_like(acc_ref)

    b = batch_ref[0, 0, :]
    cols = lax.broadcasted_iota(jnp.int32, (b.shape[0], G), 1)
    oh = (b[:, None] == cols).astype(jnp.float32)
    acc_ref[...] += lax.dot_general(oh, h_ref[...], (((0,), (0,)), ((), ())),
                                    preferred_element_type=jnp.float32)

    @pl.when(i == pl.num_programs(0) - 1)
    def _():
        out_ref[...] = jnp.dot(acc_ref[...], Wc_ref[...],
                               preferred_element_type=jnp.float32) + bc_ref[...]


def _pool(h, batch, W_c, b_c):
    BN = 1000
    grid = N // BN
    batch3 = batch.reshape(grid, 1, BN)
    return pl.pallas_call(
        _pool_body,
        grid=(grid,),
        in_specs=[
            pl.BlockSpec((BN, H), lambda i: (i, 0)),
            pl.BlockSpec((1, 1, BN), lambda i: (i, 0, 0)),
            pl.BlockSpec((H, C), lambda i: (0, 0)),
            pl.BlockSpec((1, C), lambda i: (0, 0)),
        ],
        out_specs=pl.BlockSpec((G, C), lambda i: (0, 0)),
        out_shape=jax.ShapeDtypeStruct((G, C), jnp.float32),
        scratch_shapes=[pltpu.VMEM((G, H), jnp.float32)],
    )(h, batch3, W_c, b_c.reshape(1, C))


def kernel(x, edge_index, edge_attr, batch,
           W_ee, b_ee,
           W_le1, b_le1, W1_1, b1_1, W2_1, b2_1, g1, be1,
           W_le2, b_le2, W1_2, b1_2, W2_2, b2_2, g2, be2,
           W_le3, b_le3, W1_3, b1_3, W2_3, b2_3, g3, be3,
           W_c, b_c):
    src = edge_index[0]
    dst = edge_index[1]
    e1 = _edge_feats(edge_attr, W_ee, b_ee, W_le1, b_le1)
    e2 = _edge_feats(edge_attr, W_ee, b_ee, W_le2, b_le2)
    e3 = _edge_feats(edge_attr, W_ee, b_ee, W_le3, b_le3)
    h = x
    for e, W1, b1, W2, b2, g, be in (
            (e1, W1_1, b1_1, W2_1, b2_1, g1, be1),
            (e2, W1_2, b1_2, W2_2, b2_2, g2, be2),
            (e3, W1_3, b1_3, W2_3, b2_3, g3, be3)):
        agg = _sc_agg(h, src, dst, e)
        h = _mlp(h, agg, W1, b1, W2, b2, g, be)
    return _pool(h, batch, W_c, b_c)


# speedup vs baseline: 3.3725x; 3.3725x over previous
import functools

import jax
import jax.numpy as jnp
from jax import lax
from jax.experimental import pallas as pl
from jax.experimental.pallas import tpu as pltpu
from jax.experimental.pallas import tpu_sc as plsc

N = 10000
E = 320000
D_EDGE = 16
H = 128
C = 2
G = 64
BN_EPS = 1e-5

NC = 2
NS = 16
NW = NC * NS
EDGES_PER_TILE = E // NW
CH = 40
NCHUNK = EDGES_PER_TILE // CH
NSUPER = NCHUNK // 2
NP = 10240
ROWS_PER_TILE = NP // NS
ZR = 128
HV = H // 16
EW = H // 2


def _select_matrix(offset):
    col = lax.broadcasted_iota(jnp.int32, (H, EW), 1)
    row = lax.broadcasted_iota(jnp.int32, (H, EW), 0)
    src = (col // 16) * 32 + offset + (col % 16)
    return (row == src).astype(jnp.float32)


def _bf16_bits(x):
    u = lax.bitcast_convert_type(x, jnp.uint32)
    return (u + jnp.uint32(0x7FFF) + ((u >> 16) & jnp.uint32(1))) >> 16


def _edge_feats_body(attr_ref, Wee_ref, bee_ref, Wle_ref, ble_ref, e_ref):
    attr = attr_ref[...]
    Wc = jnp.dot(Wee_ref[...], Wle_ref[...], preferred_element_type=jnp.float32)
    bc = jnp.dot(bee_ref[...], Wle_ref[...],
                 preferred_element_type=jnp.float32) + ble_ref[...]
    eA = jnp.dot(attr, jnp.dot(Wc, _select_matrix(0),
                               preferred_element_type=jnp.float32),
                 preferred_element_type=jnp.float32) \
        + jnp.dot(bc, _select_matrix(0), preferred_element_type=jnp.float32)
    eB = jnp.dot(attr, jnp.dot(Wc, _select_matrix(16),
                               preferred_element_type=jnp.float32),
                 preferred_element_type=jnp.float32) \
        + jnp.dot(bc, _select_matrix(16), preferred_element_type=jnp.float32)
    w = _bf16_bits(eA) | (_bf16_bits(eB) << 16)
    e_ref[...] = lax.bitcast_convert_type(w, jnp.int32)


def _edge_feats(edge_attr, W_ee, b_ee, Wle, ble):
    BE = 4000
    grid = E // BE
    full = lambda shape: pl.BlockSpec(shape, lambda i: (0, 0))
    return pl.pallas_call(
        _edge_feats_body,
        grid=(grid,),
        in_specs=[
            pl.BlockSpec((BE, D_EDGE), lambda i: (i, 0)),
            full((D_EDGE, H)), full((1, H)),
            full((H, H)), full((1, H)),
        ],
        out_specs=pl.BlockSpec((BE, EW), lambda i: (i, 0)),
        out_shape=jax.ShapeDtypeStruct((E, EW), jnp.int32),
    )(edge_attr, W_ee, b_ee.reshape(1, H), Wle, ble.reshape(1, H))


def _sc_agg_body(x_hbm, src_hbm, dst_hbm, e_hbm, out_hbm,
                 src_v, dst_v, rows_v, e_v, m_v, z_v, acc,
                 sem_src, sem_dst, sem_g, sem_e):
    c = lax.axis_index("c")
    s = lax.axis_index("s")
    wid = c * NS + s

    zeros16 = jnp.zeros((16,), jnp.float32)

    def zfill(i, _):
        r = i // HV
        col = (i % HV) * 16
        z_v[r, pl.ds(col, 16)] = zeros16
        return 0

    lax.fori_loop(0, ZR * HV, zfill, 0)

    def zcopy(j, _):
        pltpu.sync_copy(z_v, acc.at[pl.ds(s * ROWS_PER_TILE + j * ZR, ZR)])
        return 0

    lax.fori_loop(0, ROWS_PER_TILE // ZR, zcopy, 0)
    plsc.subcore_barrier()

    base0 = wid * EDGES_PER_TILE

    def issue_src(ci, b):
        pltpu.async_copy(src_hbm.at[pl.ds(base0 + ci * CH, CH)], src_v[b],
                         sem_src[b])

    def issue_chunk(ci, b):
        pltpu.async_copy(x_hbm.at[src_v[b]], rows_v[b], sem_g[b])
        pltpu.async_copy(e_hbm.at[pl.ds(base0 + ci * CH, CH)], e_v[b],
                         sem_e[b])
        pltpu.async_copy(dst_hbm.at[pl.ds(base0 + ci * CH, CH)], dst_v[b],
                         sem_dst[b])

    def wait_idx(sem, ref):
        pltpu.make_async_copy(src_hbm.at[pl.ds(0, CH)], ref, sem).wait()

    def wait_row(sem, ref):
        pltpu.make_async_copy(e_hbm.at[pl.ds(0, CH)], ref, sem).wait()

    def stage(k, cur, ci, has_next, has_next2):
        nxt = 1 - cur

        @pl.when(has_next)
        def _():
            wait_idx(sem_src[nxt], src_v[nxt])
            issue_chunk(ci + 1, nxt)

        pltpu.make_async_copy(x_hbm.at[src_v[cur]], rows_v[cur],
                              sem_g[cur]).wait()
        wait_row(sem_e[cur], e_v[cur])

        @pl.when(has_next2)
        def _():
            issue_src(ci + 2, cur)

        @plsc.parallel_loop(0, CH, 1, unroll=2)
        def _compute(i):
            for g in range(4):
                w = e_v[cur][i, pl.ds(g * 16, 16)]
                a = lax.bitcast_convert_type(w << 16, jnp.float32)
                b = lax.bitcast_convert_type(w & jnp.int32(-65536), jnp.float32)
                x0 = rows_v[cur][i, pl.ds(g * 32, 16)]
                x1 = rows_v[cur][i, pl.ds(g * 32 + 16, 16)]
                m_v[cur][i, pl.ds(g * 32, 16)] = jnp.maximum(x0 + a, 0.0)
                m_v[cur][i, pl.ds(g * 32 + 16, 16)] = jnp.maximum(x1 + b, 0.0)

        wait_idx(sem_dst[cur], dst_v[cur])
        pltpu.sync_copy(m_v[cur], acc.at[dst_v[cur]], add=True)

    issue_src(0, 0)
    wait_idx(sem_src[0], src_v[0])
    issue_chunk(0, 0)
    issue_src(1, 1)

    def super_step(k, _):
        t = jnp.bool_(True)
        stage(k, 0, 2 * k, t, k < NSUPER - 1)
        stage(k, 1, 2 * k + 1, k < NSUPER - 1, k < NSUPER - 1)
        return 0

    lax.fori_loop(0, NSUPER, super_step, 0)
    plsc.subcore_barrier()
    pltpu.sync_copy(acc.at[pl.ds(s * ROWS_PER_TILE, ROWS_PER_TILE)],
                    out_hbm.at[c, pl.ds(s * ROWS_PER_TILE, ROWS_PER_TILE)])


def _sc_agg(x, src, dst, e):
    mesh = plsc.VectorSubcoreMesh(core_axis_name="c", subcore_axis_name="s",
                                  num_cores=NC, num_subcores=NS)
    idx2 = [pltpu.VMEM((CH,), jnp.int32)] * 2
    buf2 = [pltpu.VMEM((CH, H), jnp.float32)] * 2
    ebuf2 = [pltpu.VMEM((CH, EW), jnp.int32)] * 2
    sem2 = [pltpu.SemaphoreType.DMA] * 2
    k = functools.partial(
        pl.kernel,
        out_type=jax.ShapeDtypeStruct((NC, NP, H), jnp.float32),
        mesh=mesh,
        scratch_types=[
            idx2, idx2, buf2, ebuf2, buf2,
            pltpu.VMEM((ZR, H), jnp.float32),
            pltpu.VMEM_SHARED((NP, H), jnp.float32),
            sem2, sem2, sem2, sem2,
        ],
    )(_sc_agg_body)
    return k(x, src, dst, e)


def _mlp_body(x_ref, agg_ref, W1_ref, b1_ref, W2_ref, b2_ref, sc_ref, be_ref,
              out_ref):
    h = x_ref[...] + agg_ref[0] + agg_ref[1]
    h1 = jnp.maximum(jnp.dot(h, W1_ref[...], preferred_element_type=jnp.float32)
                     + b1_ref[...], 0.0)
    h2 = jnp.dot(h1, W2_ref[...], preferred_element_type=jnp.float32) + b2_ref[...]
    out_ref[...] = jnp.maximum(h2 * sc_ref[...] + be_ref[...], 0.0)


def _mlp(x, agg, W1, b1, W2, b2, g, be):
    BN = 1000
    grid = N // BN
    scale = (g / jnp.sqrt(1.0 + BN_EPS)).reshape(1, H)
    full = lambda shape: pl.BlockSpec(shape, lambda i: (0, 0))
    return pl.pallas_call(
        _mlp_body,
        grid=(grid,),
        in_specs=[
            pl.BlockSpec((BN, H), lambda i: (i, 0)),
            pl.BlockSpec((NC, BN, H), lambda i: (0, i, 0)),
            full((H, H)), full((1, H)),
            full((H, H)), full((1, H)),
            full((1, H)), full((1, H)),
        ],
        out_specs=pl.BlockSpec((BN, H), lambda i: (i, 0)),
        out_shape=jax.ShapeDtypeStruct((N, H), jnp.float32),
    )(x, agg, W1, b1.reshape(1, H), W2, b2.reshape(1, H), scale,
      be.reshape(1, H))


def _pool_body(h_ref, batch_ref, Wc_ref, bc_ref, out_ref, acc_ref):
    i = pl.program_id(0)

    @pl.when(i == 0)
    def _():
        acc_ref[...] = jnp.zeros_like(acc_ref)

    b = batch_ref[0, 0, :]
    cols = lax.broadcasted_iota(jnp.int32, (b.shape[0], G), 1)
    oh = (b[:, None] == cols).astype(jnp.float32)
    acc_ref[...] += lax.dot_general(oh, h_ref[...], (((0,), (0,)), ((), ())),
                                    preferred_element_type=jnp.float32)

    @pl.when(i == pl.num_programs(0) - 1)
    def _():
        out_ref[...] = jnp.dot(acc_ref[...], Wc_ref[...],
                               preferred_element_type=jnp.float32) + bc_ref[...]


def _pool(h, batch, W_c, b_c):
    BN = 1000
    grid = N // BN
    batch3 = batch.reshape(grid, 1, BN)
    return pl.pallas_call(
        _pool_body,
        grid=(grid,),
        in_specs=[
            pl.BlockSpec((BN, H), lambda i: (i, 0)),
            pl.BlockSpec((1, 1, BN), lambda i: (i, 0, 0)),
            pl.BlockSpec((H, C), lambda i: (0, 0)),
            pl.BlockSpec((1, C), lambda i: (0, 0)),
        ],
        out_specs=pl.BlockSpec((G, C), lambda i: (0, 0)),
        out_shape=jax.ShapeDtypeStruct((G, C), jnp.float32),
        scratch_shapes=[pltpu.VMEM((G, H), jnp.float32)],
    )(h, batch3, W_c, b_c.reshape(1, C))


def kernel(x, edge_index, edge_attr, batch,
           W_ee, b_ee,
           W_le1, b_le1, W1_1, b1_1, W2_1, b2_1, g1, be1,
           W_le2, b_le2, W1_2, b1_2, W2_2, b2_2, g2, be2,
           W_le3, b_le3, W1_3, b1_3, W2_3, b2_3, g3, be3,
           W_c, b_c):
    src = edge_index[0]
    dst = edge_index[1]
    e1 = _edge_feats(edge_attr, W_ee, b_ee, W_le1, b_le1)
    e2 = _edge_feats(edge_attr, W_ee, b_ee, W_le2, b_le2)
    e3 = _edge_feats(edge_attr, W_ee, b_ee, W_le3, b_le3)
    h = x
    for e, W1, b1, W2, b2, g, be in (
            (e1, W1_1, b1_1, W2_1, b2_1, g1, be1),
            (e2, W1_2, b1_2, W2_2, b2_2, g2, be2),
            (e3, W1_3, b1_3, W2_3, b2_3, g3, be3)):
        agg = _sc_agg(h, src, dst, e)
        h = _mlp(h, agg, W1, b1, W2, b2, g, be)
    return _pool(h, batch, W_c, b_c)


# speedup vs baseline: 3.7070x; 1.0992x over previous
import functools

import jax
import jax.numpy as jnp
from jax import lax
from jax.experimental import pallas as pl
from jax.experimental.pallas import tpu as pltpu
from jax.experimental.pallas import tpu_sc as plsc

N = 10000
E = 320000
D_EDGE = 16
H = 128
C = 2
G = 64
BN_EPS = 1e-5

NC = 2
NS = 16
NW = NC * NS
EDGES_PER_TILE = E // NW
CH = 80
NCHUNK = EDGES_PER_TILE // CH
NSUPER = (NCHUNK - 1) // 2
NP = 10240
ROWS_PER_TILE = NP // NS
ZR = 64
HV = H // 16
EW = H // 2


def _select_matrix(offset):
    col = lax.broadcasted_iota(jnp.int32, (H, EW), 1)
    row = lax.broadcasted_iota(jnp.int32, (H, EW), 0)
    src = (col // 16) * 32 + offset + (col % 16)
    return (row == src).astype(jnp.float32)


def _bf16_bits(x):
    u = lax.bitcast_convert_type(x, jnp.uint32)
    return (u + jnp.uint32(0x7FFF) + ((u >> 16) & jnp.uint32(1))) >> 16


def _edge_feats_body(attr_ref, Wee_ref, bee_ref, Wle_ref, ble_ref, e_ref):
    attr = attr_ref[...]
    Wc = jnp.dot(Wee_ref[...], Wle_ref[...], preferred_element_type=jnp.float32)
    bc = jnp.dot(bee_ref[...], Wle_ref[...],
                 preferred_element_type=jnp.float32) + ble_ref[...]
    eA = jnp.dot(attr, jnp.dot(Wc, _select_matrix(0),
                               preferred_element_type=jnp.float32),
                 preferred_element_type=jnp.float32) \
        + jnp.dot(bc, _select_matrix(0), preferred_element_type=jnp.float32)
    eB = jnp.dot(attr, jnp.dot(Wc, _select_matrix(16),
                               preferred_element_type=jnp.float32),
                 preferred_element_type=jnp.float32) \
        + jnp.dot(bc, _select_matrix(16), preferred_element_type=jnp.float32)
    w = _bf16_bits(eA) | (_bf16_bits(eB) << 16)
    e_ref[...] = lax.bitcast_convert_type(w, jnp.int32)


def _edge_feats(edge_attr, W_ee, b_ee, Wle, ble):
    BE = 4000
    grid = E // BE
    full = lambda shape: pl.BlockSpec(shape, lambda i: (0, 0))
    return pl.pallas_call(
        _edge_feats_body,
        grid=(grid,),
        in_specs=[
            pl.BlockSpec((BE, D_EDGE), lambda i: (i, 0)),
            full((D_EDGE, H)), full((1, H)),
            full((H, H)), full((1, H)),
        ],
        out_specs=pl.BlockSpec((BE, EW), lambda i: (i, 0)),
        out_shape=jax.ShapeDtypeStruct((E, EW), jnp.int32),
    )(edge_attr, W_ee, b_ee.reshape(1, H), Wle, ble.reshape(1, H))


def _sc_agg_body(x_hbm, src_hbm, dst_hbm, e_hbm, out_hbm,
                 src_v, dst_v, rows_v, e_v, acc,
                 sem_src, sem_dst, sem_g, sem_e):
    c = lax.axis_index("c")
    s = lax.axis_index("s")
    wid = c * NS + s

    zeros16 = jnp.zeros((16,), jnp.float32)

    def zfill(i, _):
        r = i // HV
        col = (i % HV) * 16
        rows_v[0][r, pl.ds(col, 16)] = zeros16
        return 0

    lax.fori_loop(0, CH * HV, zfill, 0)

    def zcopy(j, _):
        pltpu.sync_copy(rows_v[0], acc.at[pl.ds(s * ROWS_PER_TILE + j * CH, CH)])
        return 0

    lax.fori_loop(0, ROWS_PER_TILE // CH, zcopy, 0)
    plsc.subcore_barrier()

    base0 = wid * EDGES_PER_TILE

    def issue_src(ci, b):
        pltpu.async_copy(src_hbm.at[pl.ds(base0 + ci * CH, CH)], src_v[b],
                         sem_src[b])

    def issue_chunk(ci, b):
        pltpu.async_copy(x_hbm.at[src_v[b]], rows_v[b], sem_g[b])
        pltpu.async_copy(e_hbm.at[pl.ds(base0 + ci * CH, CH)], e_v[b],
                         sem_e[b])
        pltpu.async_copy(dst_hbm.at[pl.ds(base0 + ci * CH, CH)], dst_v[b],
                         sem_dst[b])

    def wait_idx(sem, ref):
        pltpu.make_async_copy(src_hbm.at[pl.ds(0, CH)], ref, sem).wait()

    def wait_row(sem, ref):
        pltpu.make_async_copy(e_hbm.at[pl.ds(0, CH)], ref, sem).wait()

    def stage(k, cur, ci, has_next, has_next2):
        nxt = 1 - cur

        @pl.when(has_next)
        def _():
            wait_idx(sem_src[nxt], src_v[nxt])
            issue_chunk(ci + 1, nxt)

        pltpu.make_async_copy(x_hbm.at[src_v[cur]], rows_v[cur],
                              sem_g[cur]).wait()
        wait_row(sem_e[cur], e_v[cur])

        @pl.when(has_next2)
        def _():
            issue_src(ci + 2, cur)

        @plsc.parallel_loop(0, CH, 1, unroll=2)
        def _compute(i):
            for g in range(4):
                w = e_v[cur][i, pl.ds(g * 16, 16)]
                a = lax.bitcast_convert_type(w << 16, jnp.float32)
                b = lax.bitcast_convert_type(w & jnp.int32(-65536), jnp.float32)
                x0 = rows_v[cur][i, pl.ds(g * 32, 16)]
                x1 = rows_v[cur][i, pl.ds(g * 32 + 16, 16)]
                rows_v[cur][i, pl.ds(g * 32, 16)] = jnp.maximum(x0 + a, 0.0)
                rows_v[cur][i, pl.ds(g * 32 + 16, 16)] = jnp.maximum(x1 + b, 0.0)

        wait_idx(sem_dst[cur], dst_v[cur])
        pltpu.sync_copy(rows_v[cur], acc.at[dst_v[cur]], add=True)

    issue_src(0, 0)
    wait_idx(sem_src[0], src_v[0])
    issue_chunk(0, 0)
    issue_src(1, 1)

    def super_step(k, _):
        t = jnp.bool_(True)
        stage(k, 0, 2 * k, t, t)
        stage(k, 1, 2 * k + 1, t, k < NSUPER - 1)
        return 0

    lax.fori_loop(0, NSUPER, super_step, 0)
    f = jnp.bool_(False)
    stage(NSUPER, 0, NCHUNK - 1, f, f)
    plsc.subcore_barrier()
    pltpu.sync_copy(acc.at[pl.ds(s * ROWS_PER_TILE, ROWS_PER_TILE)],
                    out_hbm.at[c, pl.ds(s * ROWS_PER_TILE, ROWS_PER_TILE)])


def _sc_agg(x, src, dst, e):
    mesh = plsc.VectorSubcoreMesh(core_axis_name="c", subcore_axis_name="s",
                                  num_cores=NC, num_subcores=NS)
    idx2 = [pltpu.VMEM((CH,), jnp.int32)] * 2
    buf2 = [pltpu.VMEM((CH, H), jnp.float32)] * 2
    ebuf2 = [pltpu.VMEM((CH, EW), jnp.int32)] * 2
    sem2 = [pltpu.SemaphoreType.DMA] * 2
    k = functools.partial(
        pl.kernel,
        out_type=jax.ShapeDtypeStruct((NC, NP, H), jnp.float32),
        mesh=mesh,
        scratch_types=[
            idx2, idx2, buf2, ebuf2,
            pltpu.VMEM_SHARED((NP, H), jnp.float32),
            sem2, sem2, sem2, sem2,
        ],
    )(_sc_agg_body)
    return k(x, src, dst, e)


def _mlp_body(x_ref, agg_ref, W1_ref, b1_ref, W2_ref, b2_ref, sc_ref, be_ref,
              out_ref):
    h = x_ref[...] + agg_ref[0] + agg_ref[1]
    h1 = jnp.maximum(jnp.dot(h, W1_ref[...], preferred_element_type=jnp.float32)
                     + b1_ref[...], 0.0)
    h2 = jnp.dot(h1, W2_ref[...], preferred_element_type=jnp.float32) + b2_ref[...]
    out_ref[...] = jnp.maximum(h2 * sc_ref[...] + be_ref[...], 0.0)


def _mlp(x, agg, W1, b1, W2, b2, g, be):
    BN = 1000
    grid = N // BN
    scale = (g / jnp.sqrt(1.0 + BN_EPS)).reshape(1, H)
    full = lambda shape: pl.BlockSpec(shape, lambda i: (0, 0))
    return pl.pallas_call(
        _mlp_body,
        grid=(grid,),
        in_specs=[
            pl.BlockSpec((BN, H), lambda i: (i, 0)),
            pl.BlockSpec((NC, BN, H), lambda i: (0, i, 0)),
            full((H, H)), full((1, H)),
            full((H, H)), full((1, H)),
            full((1, H)), full((1, H)),
        ],
        out_specs=pl.BlockSpec((BN, H), lambda i: (i, 0)),
        out_shape=jax.ShapeDtypeStruct((N, H), jnp.float32),
    )(x, agg, W1, b1.reshape(1, H), W2, b2.reshape(1, H), scale,
      be.reshape(1, H))


def _mlp_pool_body(x_ref, agg_ref, W1_ref, b1_ref, W2_ref, b2_ref, sc_ref,
                   be_ref, batch_ref, Wc_ref, bc_ref, out_ref, acc_ref):
    i = pl.program_id(0)

    @pl.when(i == 0)
    def _():
        acc_ref[...] = jnp.zeros_like(acc_ref)

    h = x_ref[...] + agg_ref[0] + agg_ref[1]
    h1 = jnp.maximum(jnp.dot(h, W1_ref[...], preferred_element_type=jnp.float32)
                     + b1_ref[...], 0.0)
    h2 = jnp.dot(h1, W2_ref[...], preferred_element_type=jnp.float32) + b2_ref[...]
    h3 = jnp.maximum(h2 * sc_ref[...] + be_ref[...], 0.0)
    b = batch_ref[0, 0, :]
    cols = lax.broadcasted_iota(jnp.int32, (b.shape[0], G), 1)
    oh = (b[:, None] == cols).astype(jnp.float32)
    acc_ref[...] += lax.dot_general(oh, h3, (((0,), (0,)), ((), ())),
                                    preferred_element_type=jnp.float32)

    @pl.when(i == pl.num_programs(0) - 1)
    def _():
        out_ref[...] = jnp.dot(acc_ref[...], Wc_ref[...],
                               preferred_element_type=jnp.float32) + bc_ref[...]


def _mlp_pool(x, agg, W1, b1, W2, b2, g, be, batch, W_c, b_c):
    BN = 1000
    grid = N // BN
    batch3 = batch.reshape(grid, 1, BN)
    scale = (g / jnp.sqrt(1.0 + BN_EPS)).reshape(1, H)
    full = lambda shape: pl.BlockSpec(shape, lambda i: (0, 0))
    return pl.pallas_call(
        _mlp_pool_body,
        grid=(grid,),
        in_specs=[
            pl.BlockSpec((BN, H), lambda i: (i, 0)),
            pl.BlockSpec((NC, BN, H), lambda i: (0, i, 0)),
            full((H, H)), full((1, H)),
            full((H, H)), full((1, H)),
            full((1, H)), full((1, H)),
            pl.BlockSpec((1, 1, BN), lambda i: (i, 0, 0)),
            full((H, C)), full((1, C)),
        ],
        out_specs=pl.BlockSpec((G, C), lambda i: (0, 0)),
        out_shape=jax.ShapeDtypeStruct((G, C), jnp.float32),
        scratch_shapes=[pltpu.VMEM((G, H), jnp.float32)],
    )(x, agg, W1, b1.reshape(1, H), W2, b2.reshape(1, H), scale,
      be.reshape(1, H), batch3, W_c, b_c.reshape(1, C))


def kernel(x, edge_index, edge_attr, batch,
           W_ee, b_ee,
           W_le1, b_le1, W1_1, b1_1, W2_1, b2_1, g1, be1,
           W_le2, b_le2, W1_2, b1_2, W2_2, b2_2, g2, be2,
           W_le3, b_le3, W1_3, b1_3, W2_3, b2_3, g3, be3,
           W_c, b_c):
    src = edge_index[0]
    dst = edge_index[1]
    e1 = _edge_feats(edge_attr, W_ee, b_ee, W_le1, b_le1)
    e2 = _edge_feats(edge_attr, W_ee, b_ee, W_le2, b_le2)
    e3 = _edge_feats(edge_attr, W_ee, b_ee, W_le3, b_le3)
    h = x
    for e, W1, b1, W2, b2, g, be in (
            (e1, W1_1, b1_1, W2_1, b2_1, g1, be1),
            (e2, W1_2, b1_2, W2_2, b2_2, g2, be2)):
        agg = _sc_agg(h, src, dst, e)
        h = _mlp(h, agg, W1, b1, W2, b2, g, be)
    agg = _sc_agg(h, src, dst, e3)
    return _mlp_pool(h, agg, W1_3, b1_3, W2_3, b2_3, g3, be3,
                     batch, W_c, b_c)
